# baseline (speedup 1.0000x reference)
import functools

import jax
import jax.numpy as jnp
from jax import lax
from jax.experimental import pallas as pl
from jax.experimental.pallas import tpu as pltpu
from jax.experimental.pallas import tpu_sc as plsc

_NUM_USERS = 20000
_NUM_ITEMS = 30000
_N = _NUM_USERS + _NUM_ITEMS
_D = 32
_G = 4
_B = 4096
_E = 1600000

_NC, _NS, _L = 2, 16, 16
_NW = _NC * _NS
_CHUNK = 128
_SUPER = 7
_RING = 6
_PF = 3
_NBLK = 392
_NSB = _NBLK // _SUPER
_EPAD = _NW * _NBLK * _CHUNK
_NPAD = 50048
_RPT = _NPAD // _NS

_mesh = plsc.VectorSubcoreMesh(
    core_axis_name="c", subcore_axis_name="s", num_cores=_NC, num_subcores=_NS)


@functools.partial(
    pl.kernel,
    out_type=jax.ShapeDtypeStruct((_NC, _NPAD, _D), jnp.float32),
    mesh=_mesh,
    compiler_params=pltpu.CompilerParams(use_tc_tiling_on_sc=False),
    scratch_types=[
        pltpu.VMEM((_SUPER, _CHUNK), jnp.int32),
        pltpu.VMEM((_SUPER, _CHUNK), jnp.int32),
        pltpu.VMEM((_SUPER, _CHUNK), jnp.float32),
        pltpu.VMEM((_RING, _CHUNK, _D), jnp.float32),
        pltpu.VMEM_SHARED((_NPAD, _D), jnp.float32),
        pltpu.SemaphoreType.DMA((_RING,)),
        pltpu.SemaphoreType.DMA((_RING,)),
    ],
)
def _spmm(rows_hbm, cols_hbm, vals_hbm, table_hbm, zeros_hbm, out_hbm,
          idxr_v, idxc_v, vals_v, gbuf, acc, sem_g, sem_s):
    cid = lax.axis_index("c")
    sid = lax.axis_index("s")
    w = cid * _NS + sid

    r0 = sid * _RPT
    pltpu.sync_copy(zeros_hbm.at[pl.ds(r0, _RPT)], acc.at[pl.ds(r0, _RPT)])
    plsc.subcore_barrier()

    def _gather(k):
        return pltpu.make_async_copy(
            table_hbm.at[idxc_v.at[k]], gbuf.at[k % _RING],
            sem_g.at[k % _RING])

    def _scatter(k):
        return pltpu.make_async_copy(
            gbuf.at[k % _RING], acc.at[idxr_v.at[k]], sem_s.at[k % _RING])

    def superblock(sb, _):
        pltpu.sync_copy(rows_hbm.at[w, sb], idxr_v)
        pltpu.sync_copy(cols_hbm.at[w, sb], idxc_v)
        pltpu.sync_copy(vals_hbm.at[w, sb], vals_v)
        for k in range(_PF):
            _gather(k).start()
        for k in range(_SUPER):
            _gather(k).wait()

            def scale(grp, _, k=k):
                r = k % _RING
                vv = vals_v[k, pl.ds(grp * _L, _L)]
                for e2 in range(_L):
                    v = vv.at[jnp.full((_L,), e2, jnp.int32)].get(
                        mode="promise_in_bounds")
                    e = grp * _L + e2
                    g0 = gbuf[r, e, pl.ds(0, _L)]
                    g1 = gbuf[r, e, pl.ds(_L, _L)]
                    gbuf[r, e, pl.ds(0, _L)] = g0 * v
                    gbuf[r, e, pl.ds(_L, _L)] = g1 * v
                return 0

            lax.fori_loop(0, _CHUNK // _L, scale, 0)
            _scatter(k).start(add=True)
            nk = k + _PF
            if nk < _SUPER:
                if nk >= _RING:
                    _scatter(nk - _RING).wait()
                _gather(nk).start()
        for k in range(_SUPER - _RING, _SUPER):
            _scatter(k).wait()
        return 0

    lax.fori_loop(0, _NSB, superblock, 0)

    plsc.subcore_barrier()
    pltpu.sync_copy(acc.at[pl.ds(r0, _RPT)], out_hbm.at[cid, pl.ds(r0, _RPT)])


@functools.partial(
    pl.kernel,
    out_type=jax.ShapeDtypeStruct((_B, _D), jnp.float32),
    mesh=_mesh,
    compiler_params=pltpu.CompilerParams(use_tc_tiling_on_sc=False),
    scratch_types=[
        pltpu.VMEM((_CHUNK,), jnp.int32),
        pltpu.VMEM((_CHUNK, _D), jnp.float32),
    ],
)
def _gather_rows(table_hbm, idx_hbm, out_hbm, idx_v, gbuf):
    cid = lax.axis_index("c")
    sid = lax.axis_index("s")
    w = cid * _NS + sid
    pltpu.sync_copy(idx_hbm.at[w, 0], idx_v)
    pltpu.sync_copy(table_hbm.at[idx_v], gbuf)
    pltpu.sync_copy(gbuf, out_hbm.at[pl.ds(w * _CHUNK, _CHUNK)])


_BLK = 1000


def _dense1_body(emb_ref, p_ref, fcw_ref, fcb_ref, fgw_ref, fgb_ref,
                 side_ref, oh_ref, s0_ref, s1_ref, s2_ref, s3_ref):
    side = p_ref[0] + p_ref[1]
    x = emb_ref[...] + side
    t = jnp.dot(x, fcw_ref[...], preferred_element_type=jnp.float32)
    t = t + fcb_ref[...]
    t = jnp.where(t >= 0, t, 0.01 * t)
    sc = jnp.dot(t, fgw_ref[...], preferred_element_type=jnp.float32)
    sc = sc + fgb_ref[...]
    amax = jnp.max(sc, axis=1, keepdims=True)
    rows = pl.program_id(0) * _BLK + lax.broadcasted_iota(
        jnp.int32, (_BLK, 1), 0)
    oh = jnp.where(rows < _NUM_USERS,
                   (sc == amax).astype(jnp.float32),
                   jnp.float32(1.0))
    side_ref[...] = side
    oh_ref[...] = oh
    s0_ref[...] = oh[:, 0:1] * side
    s1_ref[...] = oh[:, 1:2] * side
    s2_ref[...] = oh[:, 2:3] * side
    s3_ref[...] = oh[:, 3:4] * side


def _dense1(all_emb, partials, fc_W, fc_b, fcg_W, fcg_b):
    f = jnp.float32
    return pl.pallas_call(
        _dense1_body,
        grid=(_N // _BLK,),
        in_specs=[
            pl.BlockSpec((_BLK, _D), lambda i: (i, 0)),
            pl.BlockSpec((_NC, _BLK, _D), lambda i: (0, i, 0)),
            pl.BlockSpec((_D, _D), lambda i: (0, 0)),
            pl.BlockSpec((1, _D), lambda i: (0, 0)),
            pl.BlockSpec((_D, _G), lambda i: (0, 0)),
            pl.BlockSpec((1, _G), lambda i: (0, 0)),
        ],
        out_specs=[
            pl.BlockSpec((_BLK, _D), lambda i: (i, 0)),
            pl.BlockSpec((_BLK, _G), lambda i: (i, 0)),
            pl.BlockSpec((_BLK, _D), lambda i: (i, 0)),
            pl.BlockSpec((_BLK, _D), lambda i: (i, 0)),
            pl.BlockSpec((_BLK, _D), lambda i: (i, 0)),
            pl.BlockSpec((_BLK, _D), lambda i: (i, 0)),
        ],
        out_shape=[
            jax.ShapeDtypeStruct((_N, _D), f),
            jax.ShapeDtypeStruct((_N, _G), f),
            jax.ShapeDtypeStruct((_N, _D), f),
            jax.ShapeDtypeStruct((_N, _D), f),
            jax.ShapeDtypeStruct((_N, _D), f),
            jax.ShapeDtypeStruct((_N, _D), f),
        ],
    )(all_emb, partials, fc_W, fc_b[None, :], fcg_W, fcg_b[None, :])


def _dense2_body(z0_ref, z1_ref, z2_ref, z3_ref, oh_ref,
                 l0_ref, l1_ref, l2_ref, l3_ref, ls_ref):
    oh = oh_ref[...]
    outs = []
    for g, zref in enumerate((z0_ref, z1_ref, z2_ref, z3_ref)):
        zs = zref[0] + zref[1]
        outs.append(oh[:, g:g + 1] * zs)
    l0_ref[...] = outs[0]
    l1_ref[...] = outs[1]
    l2_ref[...] = outs[2]
    l3_ref[...] = outs[3]
    ls_ref[...] = outs[0] + outs[1] + outs[2] + outs[3]


def _dense2(z, oh):
    f = jnp.float32
    return pl.pallas_call(
        _dense2_body,
        grid=(_N // _BLK,),
        in_specs=[pl.BlockSpec((_NC, _BLK, _D), lambda i: (0, i, 0))] * 4
        + [pl.BlockSpec((_BLK, _G), lambda i: (i, 0))],
        out_specs=[pl.BlockSpec((_BLK, _D), lambda i: (i, 0))] * 5,
        out_shape=[jax.ShapeDtypeStruct((_N, _D), f)] * 5,
    )(*z, oh)


def _dense3_body(side_ref, ls_ref, y0_ref, y1_ref, y2_ref, y3_ref, oh_ref,
                 out_ref):
    oh = oh_ref[...]
    acc = 4.0 * side_ref[...] + ls_ref[...]
    for g, yref in enumerate((y0_ref, y1_ref, y2_ref, y3_ref)):
        acc = acc + oh[:, g:g + 1] * (yref[0] + yref[1])
    out_ref[...] = 0.2 * acc


def _dense3(side, l1sum, y, oh):
    return pl.pallas_call(
        _dense3_body,
        grid=(_N // _BLK,),
        in_specs=[pl.BlockSpec((_BLK, _D), lambda i: (i, 0))] * 2
        + [pl.BlockSpec((_NC, _BLK, _D), lambda i: (0, i, 0))] * 4
        + [pl.BlockSpec((_BLK, _G), lambda i: (i, 0))],
        out_specs=pl.BlockSpec((_BLK, _D), lambda i: (i, 0)),
        out_shape=jax.ShapeDtypeStruct((_N, _D), jnp.float32),
    )(side, l1sum, *y, oh)


def _dot_body(u_ref, v_ref, o_ref):
    o_ref[...] = jnp.sum(u_ref[...] * v_ref[...], axis=1, keepdims=True)


def _rowdot(u, v):
    return pl.pallas_call(
        _dot_body,
        grid=(1,),
        in_specs=[pl.BlockSpec((_B, _D), lambda i: (0, 0))] * 2,
        out_specs=pl.BlockSpec((_B, 1), lambda i: (0, 0)),
        out_shape=jax.ShapeDtypeStruct((_B, 1), jnp.float32),
    )(u, v)


def kernel(users, items, edge_index, graph_vals, emb_user, emb_item,
           fc_W, fc_b, fcg_W, fcg_b):
    f = jnp.float32
    all_emb = jnp.concatenate([emb_user, emb_item], axis=0)

    pad = _EPAD - _E
    rows = jnp.pad(edge_index[0], (0, pad)).reshape(_NW, _NSB, _SUPER, _CHUNK)
    cols = jnp.pad(edge_index[1], (0, pad)).reshape(_NW, _NSB, _SUPER, _CHUNK)
    vals = jnp.pad(graph_vals, (0, pad)).reshape(_NW, _NSB, _SUPER, _CHUNK)
    zeros = jnp.zeros((_NPAD, _D), f)

    p_side = _spmm(rows, cols, vals, all_emb, zeros)
    side, oh, s0, s1, s2, s3 = _dense1(all_emb, p_side, fc_W, fc_b,
                                       fcg_W, fcg_b)
    z = [_spmm(rows, cols, vals, s, zeros) for s in (s0, s1, s2, s3)]
    l0, l1, l2, l3, l1sum = _dense2(z, oh)
    y = [_spmm(rows, cols, vals, t, zeros) for t in (l0, l1, l2, l3)]
    all_out = _dense3(side, l1sum, y, oh)

    uidx = users.astype(jnp.int32).reshape(_NW, 1, _CHUNK)
    iidx = (items.astype(jnp.int32) + _NUM_USERS).reshape(_NW, 1, _CHUNK)
    u = _gather_rows(all_out, uidx)
    v = _gather_rows(all_out, iidx)
    return _rowdot(u, v).reshape(_B)


# speedup vs baseline: 1.1643x; 1.1643x over previous
import functools

import jax
import jax.numpy as jnp
from jax import lax
from jax.experimental import pallas as pl
from jax.experimental.pallas import tpu as pltpu
from jax.experimental.pallas import tpu_sc as plsc

_NUM_USERS = 20000
_NUM_ITEMS = 30000
_N = _NUM_USERS + _NUM_ITEMS
_D = 32
_G = 4
_B = 4096
_E = 1600000

_NC, _NS, _L = 2, 16, 16
_NW = _NC * _NS
_CHUNK = 128
_SUPER = 14
_RING = 6
_PF = 3
_NBLK = 392
_NSB = _NBLK // _SUPER
_EPAD = _NW * _NBLK * _CHUNK
_NPAD = 50048
_RPT = _NPAD // _NS

_mesh = plsc.VectorSubcoreMesh(
    core_axis_name="c", subcore_axis_name="s", num_cores=_NC, num_subcores=_NS)


@functools.partial(
    pl.kernel,
    out_type=jax.ShapeDtypeStruct((_NC, _NPAD, _D), jnp.float32),
    mesh=_mesh,
    compiler_params=pltpu.CompilerParams(use_tc_tiling_on_sc=False),
    scratch_types=[
        pltpu.VMEM((_SUPER, _CHUNK), jnp.int32),
        pltpu.VMEM((_SUPER, _CHUNK), jnp.int32),
        pltpu.VMEM((_SUPER, _CHUNK), jnp.float32),
        pltpu.VMEM((_RING, _CHUNK, _D), jnp.float32),
        pltpu.VMEM_SHARED((_NPAD, _D), jnp.float32),
        pltpu.SemaphoreType.DMA((_RING,)),
        pltpu.SemaphoreType.DMA((_RING,)),
    ],
)
def _spmm(rows_hbm, cols_hbm, vals_hbm, table_hbm, zeros_hbm, out_hbm,
          idxr_v, idxc_v, vals_v, gbuf, acc, sem_g, sem_s):
    cid = lax.axis_index("c")
    sid = lax.axis_index("s")
    w = cid * _NS + sid

    r0 = sid * _RPT
    pltpu.sync_copy(zeros_hbm.at[pl.ds(r0, _RPT)], acc.at[pl.ds(r0, _RPT)])
    plsc.subcore_barrier()

    def _gather(k):
        return pltpu.make_async_copy(
            table_hbm.at[idxc_v.at[k]], gbuf.at[k % _RING],
            sem_g.at[k % _RING])

    def _scatter(k):
        return pltpu.make_async_copy(
            gbuf.at[k % _RING], acc.at[idxr_v.at[k]], sem_s.at[k % _RING])

    def superblock(sb, _):
        pltpu.sync_copy(rows_hbm.at[w, sb], idxr_v)
        pltpu.sync_copy(cols_hbm.at[w, sb], idxc_v)
        pltpu.sync_copy(vals_hbm.at[w, sb], vals_v)
        for k in range(_PF):
            _gather(k).start()
        for k in range(_SUPER):
            _gather(k).wait()

            def scale(grp, _, k=k):
                r = k % _RING
                vv = vals_v[k, pl.ds(grp * _L, _L)]
                for e2 in range(_L):
                    v = vv.at[jnp.full((_L,), e2, jnp.int32)].get(
                        mode="promise_in_bounds")
                    e = grp * _L + e2
                    g0 = gbuf[r, e, pl.ds(0, _L)]
                    g1 = gbuf[r, e, pl.ds(_L, _L)]
                    gbuf[r, e, pl.ds(0, _L)] = g0 * v
                    gbuf[r, e, pl.ds(_L, _L)] = g1 * v
                return 0

            lax.fori_loop(0, _CHUNK // _L, scale, 0)
            _scatter(k).start(add=True)
            nk = k + _PF
            if nk < _SUPER:
                if nk >= _RING:
                    _scatter(nk - _RING).wait()
                _gather(nk).start()
        for k in range(_SUPER - _RING, _SUPER):
            _scatter(k).wait()
        return 0

    lax.fori_loop(0, _NSB, superblock, 0)

    plsc.subcore_barrier()
    pltpu.sync_copy(acc.at[pl.ds(r0, _RPT)], out_hbm.at[cid, pl.ds(r0, _RPT)])


@functools.partial(
    pl.kernel,
    out_type=jax.ShapeDtypeStruct((_B, _D), jnp.float32),
    mesh=_mesh,
    compiler_params=pltpu.CompilerParams(use_tc_tiling_on_sc=False),
    scratch_types=[
        pltpu.VMEM((_CHUNK,), jnp.int32),
        pltpu.VMEM((_CHUNK, _D), jnp.float32),
    ],
)
def _gather_rows(table_hbm, idx_hbm, out_hbm, idx_v, gbuf):
    cid = lax.axis_index("c")
    sid = lax.axis_index("s")
    w = cid * _NS + sid
    pltpu.sync_copy(idx_hbm.at[w, 0], idx_v)
    pltpu.sync_copy(table_hbm.at[idx_v], gbuf)
    pltpu.sync_copy(gbuf, out_hbm.at[pl.ds(w * _CHUNK, _CHUNK)])


_BLK = 1000


def _dense1_body(emb_ref, p_ref, fcw_ref, fcb_ref, fgw_ref, fgb_ref,
                 side_ref, oh_ref, s0_ref, s1_ref, s2_ref, s3_ref):
    side = p_ref[0] + p_ref[1]
    x = emb_ref[...] + side
    t = jnp.dot(x, fcw_ref[...], preferred_element_type=jnp.float32)
    t = t + fcb_ref[...]
    t = jnp.where(t >= 0, t, 0.01 * t)
    sc = jnp.dot(t, fgw_ref[...], preferred_element_type=jnp.float32)
    sc = sc + fgb_ref[...]
    amax = jnp.max(sc, axis=1, keepdims=True)
    rows = pl.program_id(0) * _BLK + lax.broadcasted_iota(
        jnp.int32, (_BLK, 1), 0)
    oh = jnp.where(rows < _NUM_USERS,
                   (sc == amax).astype(jnp.float32),
                   jnp.float32(1.0))
    side_ref[...] = side
    oh_ref[...] = oh
    s0_ref[...] = oh[:, 0:1] * side
    s1_ref[...] = oh[:, 1:2] * side
    s2_ref[...] = oh[:, 2:3] * side
    s3_ref[...] = oh[:, 3:4] * side


def _dense1(all_emb, partials, fc_W, fc_b, fcg_W, fcg_b):
    f = jnp.float32
    return pl.pallas_call(
        _dense1_body,
        grid=(_N // _BLK,),
        in_specs=[
            pl.BlockSpec((_BLK, _D), lambda i: (i, 0)),
            pl.BlockSpec((_NC, _BLK, _D), lambda i: (0, i, 0)),
            pl.BlockSpec((_D, _D), lambda i: (0, 0)),
            pl.BlockSpec((1, _D), lambda i: (0, 0)),
            pl.BlockSpec((_D, _G), lambda i: (0, 0)),
            pl.BlockSpec((1, _G), lambda i: (0, 0)),
        ],
        out_specs=[
            pl.BlockSpec((_BLK, _D), lambda i: (i, 0)),
            pl.BlockSpec((_BLK, _G), lambda i: (i, 0)),
            pl.BlockSpec((_BLK, _D), lambda i: (i, 0)),
            pl.BlockSpec((_BLK, _D), lambda i: (i, 0)),
            pl.BlockSpec((_BLK, _D), lambda i: (i, 0)),
            pl.BlockSpec((_BLK, _D), lambda i: (i, 0)),
        ],
        out_shape=[
            jax.ShapeDtypeStruct((_N, _D), f),
            jax.ShapeDtypeStruct((_N, _G), f),
            jax.ShapeDtypeStruct((_N, _D), f),
            jax.ShapeDtypeStruct((_N, _D), f),
            jax.ShapeDtypeStruct((_N, _D), f),
            jax.ShapeDtypeStruct((_N, _D), f),
        ],
    )(all_emb, partials, fc_W, fc_b[None, :], fcg_W, fcg_b[None, :])


def _dense2_body(z0_ref, z1_ref, z2_ref, z3_ref, oh_ref,
                 l0_ref, l1_ref, l2_ref, l3_ref, ls_ref):
    oh = oh_ref[...]
    outs = []
    for g, zref in enumerate((z0_ref, z1_ref, z2_ref, z3_ref)):
        zs = zref[0] + zref[1]
        outs.append(oh[:, g:g + 1] * zs)
    l0_ref[...] = outs[0]
    l1_ref[...] = outs[1]
    l2_ref[...] = outs[2]
    l3_ref[...] = outs[3]
    ls_ref[...] = outs[0] + outs[1] + outs[2] + outs[3]


def _dense2(z, oh):
    f = jnp.float32
    return pl.pallas_call(
        _dense2_body,
        grid=(_N // _BLK,),
        in_specs=[pl.BlockSpec((_NC, _BLK, _D), lambda i: (0, i, 0))] * 4
        + [pl.BlockSpec((_BLK, _G), lambda i: (i, 0))],
        out_specs=[pl.BlockSpec((_BLK, _D), lambda i: (i, 0))] * 5,
        out_shape=[jax.ShapeDtypeStruct((_N, _D), f)] * 5,
    )(*z, oh)


def _dense3_body(side_ref, ls_ref, y0_ref, y1_ref, y2_ref, y3_ref, oh_ref,
                 out_ref):
    oh = oh_ref[...]
    acc = 4.0 * side_ref[...] + ls_ref[...]
    for g, yref in enumerate((y0_ref, y1_ref, y2_ref, y3_ref)):
        acc = acc + oh[:, g:g + 1] * (yref[0] + yref[1])
    out_ref[...] = 0.2 * acc


def _dense3(side, l1sum, y, oh):
    return pl.pallas_call(
        _dense3_body,
        grid=(_N // _BLK,),
        in_specs=[pl.BlockSpec((_BLK, _D), lambda i: (i, 0))] * 2
        + [pl.BlockSpec((_NC, _BLK, _D), lambda i: (0, i, 0))] * 4
        + [pl.BlockSpec((_BLK, _G), lambda i: (i, 0))],
        out_specs=pl.BlockSpec((_BLK, _D), lambda i: (i, 0)),
        out_shape=jax.ShapeDtypeStruct((_N, _D), jnp.float32),
    )(side, l1sum, *y, oh)


def _dot_body(u_ref, v_ref, o_ref):
    o_ref[...] = jnp.sum(u_ref[...] * v_ref[...], axis=1, keepdims=True)


def _rowdot(u, v):
    return pl.pallas_call(
        _dot_body,
        grid=(1,),
        in_specs=[pl.BlockSpec((_B, _D), lambda i: (0, 0))] * 2,
        out_specs=pl.BlockSpec((_B, 1), lambda i: (0, 0)),
        out_shape=jax.ShapeDtypeStruct((_B, 1), jnp.float32),
    )(u, v)


def kernel(users, items, edge_index, graph_vals, emb_user, emb_item,
           fc_W, fc_b, fcg_W, fcg_b):
    f = jnp.float32
    all_emb = jnp.concatenate([emb_user, emb_item], axis=0)

    pad = _EPAD - _E
    rows = jnp.pad(edge_index[0], (0, pad)).reshape(_NW, _NSB, _SUPER, _CHUNK)
    cols = jnp.pad(edge_index[1], (0, pad)).reshape(_NW, _NSB, _SUPER, _CHUNK)
    vals = jnp.pad(graph_vals, (0, pad)).reshape(_NW, _NSB, _SUPER, _CHUNK)
    zeros = jnp.zeros((_NPAD, _D), f)

    p_side = _spmm(rows, cols, vals, all_emb, zeros)
    side, oh, s0, s1, s2, s3 = _dense1(all_emb, p_side, fc_W, fc_b,
                                       fcg_W, fcg_b)
    z = [_spmm(rows, cols, vals, s, zeros) for s in (s0, s1, s2, s3)]
    l0, l1, l2, l3, l1sum = _dense2(z, oh)
    y = [_spmm(rows, cols, vals, t, zeros) for t in (l0, l1, l2, l3)]
    all_out = _dense3(side, l1sum, y, oh)

    uidx = users.astype(jnp.int32).reshape(_NW, 1, _CHUNK)
    iidx = (items.astype(jnp.int32) + _NUM_USERS).reshape(_NW, 1, _CHUNK)
    u = _gather_rows(all_out, uidx)
    v = _gather_rows(all_out, iidx)
    return _rowdot(u, v).reshape(_B)


# speedup vs baseline: 1.1945x; 1.0260x over previous
import functools

import jax
import jax.numpy as jnp
from jax import lax
from jax.experimental import pallas as pl
from jax.experimental.pallas import tpu as pltpu
from jax.experimental.pallas import tpu_sc as plsc

_NUM_USERS = 20000
_NUM_ITEMS = 30000
_N = _NUM_USERS + _NUM_ITEMS
_D = 32
_G = 4
_B = 4096
_E = 1600000

_NC, _NS, _L = 2, 16, 16
_NW = _NC * _NS
_CHUNK = 128
_SUPER = 14
_RING = 6
_PF = 3
_NBLK = 392
_NSB = _NBLK // _SUPER
_EPAD = _NW * _NBLK * _CHUNK
_NPAD = 50176
_RPT = _NPAD // _NS
_MW = _NPAD // _NW
_KN = 392

_mesh = plsc.VectorSubcoreMesh(
    core_axis_name="c", subcore_axis_name="s", num_cores=_NC, num_subcores=_NS)
_sc_params = pltpu.CompilerParams(use_tc_tiling_on_sc=False)
_f32 = jnp.float32


def _wid():
    return lax.axis_index("c") * _NS + lax.axis_index("s")


def _splat(vec, lane):
    return vec.at[jnp.full((_L,), lane, jnp.int32)].get(
        mode="promise_in_bounds")


@functools.partial(
    pl.kernel,
    out_type=jax.ShapeDtypeStruct((_NC, _NPAD, _D), _f32),
    mesh=_mesh,
    compiler_params=_sc_params,
    scratch_types=[
        pltpu.VMEM((_SUPER, _CHUNK), jnp.int32),
        pltpu.VMEM((_SUPER, _CHUNK), jnp.int32),
        pltpu.VMEM((_SUPER, _CHUNK), _f32),
        pltpu.VMEM((_RING, _CHUNK, _D), _f32),
        pltpu.VMEM_SHARED((_NPAD, _D), _f32),
        pltpu.SemaphoreType.DMA((_RING,)),
        pltpu.SemaphoreType.DMA((_RING,)),
    ],
)
def _spmm(rows_hbm, cols_hbm, vals_hbm, table_hbm, zeros_hbm, out_hbm,
          idxr_v, idxc_v, vals_v, gbuf, acc, sem_g, sem_s):
    cid = lax.axis_index("c")
    sid = lax.axis_index("s")
    w = cid * _NS + sid

    r0 = sid * _RPT
    pltpu.sync_copy(zeros_hbm.at[pl.ds(r0, _RPT)], acc.at[pl.ds(r0, _RPT)])
    plsc.subcore_barrier()

    def _gather(k):
        return pltpu.make_async_copy(
            table_hbm.at[idxc_v.at[k]], gbuf.at[k % _RING],
            sem_g.at[k % _RING])

    def _scatter(k):
        return pltpu.make_async_copy(
            gbuf.at[k % _RING], acc.at[idxr_v.at[k]], sem_s.at[k % _RING])

    def superblock(sb, _):
        pltpu.sync_copy(rows_hbm.at[w, sb], idxr_v)
        pltpu.sync_copy(cols_hbm.at[w, sb], idxc_v)
        pltpu.sync_copy(vals_hbm.at[w, sb], vals_v)
        for k in range(_PF):
            _gather(k).start()
        for k in range(_SUPER):
            _gather(k).wait()

            def scale(grp, _, k=k):
                r = k % _RING
                vv = vals_v[k, pl.ds(grp * _L, _L)]
                for e2 in range(_L):
                    v = _splat(vv, e2)
                    e = grp * _L + e2
                    g0 = gbuf[r, e, pl.ds(0, _L)]
                    g1 = gbuf[r, e, pl.ds(_L, _L)]
                    gbuf[r, e, pl.ds(0, _L)] = g0 * v
                    gbuf[r, e, pl.ds(_L, _L)] = g1 * v
                return 0

            lax.fori_loop(0, _CHUNK // _L, scale, 0)
            _scatter(k).start(add=True)
            nk = k + _PF
            if nk < _SUPER:
                if nk >= _RING:
                    _scatter(nk - _RING).wait()
                _gather(nk).start()
        for k in range(max(0, _SUPER - _RING), _SUPER):
            _scatter(k).wait()
        return 0

    lax.fori_loop(0, _NSB, superblock, 0)

    plsc.subcore_barrier()
    pltpu.sync_copy(acc.at[pl.ds(r0, _RPT)], out_hbm.at[cid, pl.ds(r0, _RPT)])


@functools.partial(
    pl.kernel,
    out_type=[jax.ShapeDtypeStruct((_NPAD, _D), _f32)] * 5,
    mesh=_mesh,
    compiler_params=_sc_params,
    scratch_types=[
        pltpu.VMEM((_KN, _D), _f32),
        pltpu.VMEM((_KN, _D), _f32),
        pltpu.VMEM((_KN * _G,), _f32),
        pltpu.VMEM((_KN, _D), _f32),
        pltpu.VMEM((_KN, _D), _f32),
        pltpu.VMEM((_KN, _D), _f32),
        pltpu.VMEM((_KN, _D), _f32),
    ],
)
def _mask1(p_hbm, oh_hbm, side_hbm, o0, o1, o2, o3, p0v, p1v, ohv,
           s0v, s1v, s2v, s3v):
    w = _wid()

    def chunk(c, _):
        n0 = w * _MW + c * _KN
        pltpu.sync_copy(p_hbm.at[0, pl.ds(n0, _KN)], p0v)
        pltpu.sync_copy(p_hbm.at[1, pl.ds(n0, _KN)], p1v)
        pltpu.sync_copy(oh_hbm.at[pl.ds(n0 * _G, _KN * _G)], ohv)

        def node(n, _):
            a0 = p0v[n, pl.ds(0, _L)] + p1v[n, pl.ds(0, _L)]
            a1 = p0v[n, pl.ds(_L, _L)] + p1v[n, pl.ds(_L, _L)]
            p0v[n, pl.ds(0, _L)] = a0
            p0v[n, pl.ds(_L, _L)] = a1
            ov = ohv[pl.ds((n // 4) * _L, _L)]
            lane0 = _G * (n % 4)
            for g, sv in enumerate((s0v, s1v, s2v, s3v)):
                m = _splat(ov, lane0 + g)
                sv[n, pl.ds(0, _L)] = a0 * m
                sv[n, pl.ds(_L, _L)] = a1 * m
            return 0

        lax.fori_loop(0, _KN, node, 0)
        pltpu.sync_copy(p0v, side_hbm.at[pl.ds(n0, _KN)])
        for sv, oref in ((s0v, o0), (s1v, o1), (s2v, o2), (s3v, o3)):
            pltpu.sync_copy(sv, oref.at[pl.ds(n0, _KN)])
        return 0

    lax.fori_loop(0, _MW // _KN, chunk, 0)


@functools.partial(
    pl.kernel,
    out_type=[jax.ShapeDtypeStruct((_NPAD, _D), _f32)] * 5,
    mesh=_mesh,
    compiler_params=_sc_params,
    scratch_types=[
        pltpu.VMEM((_KN, _D), _f32),
        pltpu.VMEM((_KN, _D), _f32),
        pltpu.VMEM((_KN * _G,), _f32),
        pltpu.VMEM((_KN, _D), _f32),
    ],
)
def _mask2(z0, z1, z2, z3, oh_hbm, l0, l1, l2, l3, ls_hbm, zav, zbv, ohv,
           lsv):
    w = _wid()

    def chunk(c, _):
        n0 = w * _MW + c * _KN
        pltpu.sync_copy(oh_hbm.at[pl.ds(n0 * _G, _KN * _G)], ohv)
        for g, (zg, lout) in enumerate(((z0, l0), (z1, l1), (z2, l2),
                                        (z3, l3))):
            pltpu.sync_copy(zg.at[0, pl.ds(n0, _KN)], zav)
            pltpu.sync_copy(zg.at[1, pl.ds(n0, _KN)], zbv)

            def node(n, _, g=g):
                a0 = zav[n, pl.ds(0, _L)] + zbv[n, pl.ds(0, _L)]
                a1 = zav[n, pl.ds(_L, _L)] + zbv[n, pl.ds(_L, _L)]
                ov = ohv[pl.ds((n // 4) * _L, _L)]
                m = _splat(ov, _G * (n % 4) + g)
                r0 = a0 * m
                r1 = a1 * m
                zav[n, pl.ds(0, _L)] = r0
                zav[n, pl.ds(_L, _L)] = r1
                if g == 0:
                    lsv[n, pl.ds(0, _L)] = r0
                    lsv[n, pl.ds(_L, _L)] = r1
                else:
                    lsv[n, pl.ds(0, _L)] = lsv[n, pl.ds(0, _L)] + r0
                    lsv[n, pl.ds(_L, _L)] = lsv[n, pl.ds(_L, _L)] + r1
                return 0

            lax.fori_loop(0, _KN, node, 0)
            pltpu.sync_copy(zav, lout.at[pl.ds(n0, _KN)])
        pltpu.sync_copy(lsv, ls_hbm.at[pl.ds(n0, _KN)])
        return 0

    lax.fori_loop(0, _MW // _KN, chunk, 0)


@functools.partial(
    pl.kernel,
    out_type=jax.ShapeDtypeStruct((_NPAD, _D), _f32),
    mesh=_mesh,
    compiler_params=_sc_params,
    scratch_types=[
        pltpu.VMEM((_KN, _D), _f32),
        pltpu.VMEM((_KN, _D), _f32),
        pltpu.VMEM((_KN, _D), _f32),
        pltpu.VMEM((_KN, _D), _f32),
        pltpu.VMEM((_KN * _G,), _f32),
    ],
)
def _mask3(side_hbm, ls_hbm, y0, y1, y2, y3, oh_hbm, out_hbm, av, bv, zav,
           zbv, ohv):
    w = _wid()

    def chunk(c, _):
        n0 = w * _MW + c * _KN
        pltpu.sync_copy(side_hbm.at[pl.ds(n0, _KN)], av)
        pltpu.sync_copy(ls_hbm.at[pl.ds(n0, _KN)], bv)
        pltpu.sync_copy(oh_hbm.at[pl.ds(n0 * _G, _KN * _G)], ohv)

        def base_node(n, _):
            for h in range(2):
                s = av[n, pl.ds(h * _L, _L)]
                t = bv[n, pl.ds(h * _L, _L)]
                av[n, pl.ds(h * _L, _L)] = 4.0 * s + t
            return 0

        lax.fori_loop(0, _KN, base_node, 0)
        for g, yg in enumerate((y0, y1, y2, y3)):
            pltpu.sync_copy(yg.at[0, pl.ds(n0, _KN)], zav)
            pltpu.sync_copy(yg.at[1, pl.ds(n0, _KN)], zbv)

            def node(n, _, g=g):
                ov = ohv[pl.ds((n // 4) * _L, _L)]
                m = _splat(ov, _G * (n % 4) + g)
                for h in range(2):
                    y = zav[n, pl.ds(h * _L, _L)] + zbv[n, pl.ds(h * _L, _L)]
                    av[n, pl.ds(h * _L, _L)] = (
                        av[n, pl.ds(h * _L, _L)] + m * y)
                return 0

            lax.fori_loop(0, _KN, node, 0)

        def fin(n, _):
            for h in range(2):
                av[n, pl.ds(h * _L, _L)] = 0.2 * av[n, pl.ds(h * _L, _L)]
            return 0

        lax.fori_loop(0, _KN, fin, 0)
        pltpu.sync_copy(av, out_hbm.at[pl.ds(n0, _KN)])
        return 0

    lax.fori_loop(0, _MW // _KN, chunk, 0)


@functools.partial(
    pl.kernel,
    out_type=jax.ShapeDtypeStruct((_B, _D), _f32),
    mesh=_mesh,
    compiler_params=_sc_params,
    scratch_types=[
        pltpu.VMEM((_CHUNK,), jnp.int32),
        pltpu.VMEM((_CHUNK, _D), _f32),
    ],
)
def _gather_rows(table_hbm, idx_hbm, out_hbm, idx_v, gbuf):
    w = _wid()
    pltpu.sync_copy(idx_hbm.at[w, 0], idx_v)
    pltpu.sync_copy(table_hbm.at[idx_v], gbuf)
    pltpu.sync_copy(gbuf, out_hbm.at[pl.ds(w * _CHUNK, _CHUNK)])


_BLK = 6272


def _dense1_body(emb_ref, p_ref, fcw_ref, fcb_ref, fgw_ref, fgb_ref, oh_ref):
    side = p_ref[0] + p_ref[1]
    x = emb_ref[...] + side
    t = jnp.dot(x, fcw_ref[...], preferred_element_type=_f32)
    t = t + fcb_ref[...]
    t = jnp.where(t >= 0, t, 0.01 * t)
    sc = jnp.dot(t, fgw_ref[...], preferred_element_type=_f32)
    sc = sc + fgb_ref[...]
    amax = jnp.max(sc, axis=1, keepdims=True)
    rows = pl.program_id(0) * _BLK + lax.broadcasted_iota(
        jnp.int32, (_BLK, 1), 0)
    oh_ref[...] = jnp.where(rows < _NUM_USERS,
                            (sc == amax).astype(_f32), _f32(1.0))


def _dense1(all_emb, partials, fc_W, fc_b, fcg_W, fcg_b):
    return pl.pallas_call(
        _dense1_body,
        grid=(_NPAD // _BLK,),
        in_specs=[
            pl.BlockSpec((_BLK, _D), lambda i: (i, 0)),
            pl.BlockSpec((_NC, _BLK, _D), lambda i: (0, i, 0)),
            pl.BlockSpec((_D, _D), lambda i: (0, 0)),
            pl.BlockSpec((1, _D), lambda i: (0, 0)),
            pl.BlockSpec((_D, _G), lambda i: (0, 0)),
            pl.BlockSpec((1, _G), lambda i: (0, 0)),
        ],
        out_specs=pl.BlockSpec((_BLK, _G), lambda i: (i, 0)),
        out_shape=jax.ShapeDtypeStruct((_NPAD, _G), _f32),
    )(all_emb, partials, fc_W, fc_b[None, :], fcg_W, fcg_b[None, :])


def _dot_body(u_ref, v_ref, o_ref):
    o_ref[...] = jnp.sum(u_ref[...] * v_ref[...], axis=1, keepdims=True)


def _rowdot(u, v):
    return pl.pallas_call(
        _dot_body,
        grid=(1,),
        in_specs=[pl.BlockSpec((_B, _D), lambda i: (0, 0))] * 2,
        out_specs=pl.BlockSpec((_B, 1), lambda i: (0, 0)),
        out_shape=jax.ShapeDtypeStruct((_B, 1), _f32),
    )(u, v)


def kernel(users, items, edge_index, graph_vals, emb_user, emb_item,
           fc_W, fc_b, fcg_W, fcg_b):
    all_emb = jnp.concatenate([emb_user, emb_item], axis=0)
    all_emb = jnp.pad(all_emb, ((0, _NPAD - _N), (0, 0)))

    pad = _EPAD - _E
    rows = jnp.pad(edge_index[0], (0, pad)).reshape(_NW, _NSB, _SUPER, _CHUNK)
    cols = jnp.pad(edge_index[1], (0, pad)).reshape(_NW, _NSB, _SUPER, _CHUNK)
    vals = jnp.pad(graph_vals, (0, pad)).reshape(_NW, _NSB, _SUPER, _CHUNK)
    zeros = jnp.zeros((_NPAD, _D), _f32)

    p_side = _spmm(rows, cols, vals, all_emb, zeros)
    oh = _dense1(all_emb, p_side, fc_W, fc_b, fcg_W, fcg_b)
    ohf = oh.reshape(-1)
    side, s0, s1, s2, s3 = _mask1(p_side, ohf)
    z = [_spmm(rows, cols, vals, s, zeros) for s in (s0, s1, s2, s3)]
    l0, l1, l2, l3, l1sum = _mask2(*z, ohf)
    y = [_spmm(rows, cols, vals, t, zeros) for t in (l0, l1, l2, l3)]
    all_out = _mask3(side, l1sum, *y, ohf)

    uidx = users.astype(jnp.int32).reshape(_NW, 1, _CHUNK)
    iidx = (items.astype(jnp.int32) + _NUM_USERS).reshape(_NW, 1, _CHUNK)
    u = _gather_rows(all_out, uidx)
    v = _gather_rows(all_out, iidx)
    return _rowdot(u, v).reshape(_B)


# speedup vs baseline: 1.2866x; 1.0771x over previous
import functools

import jax
import jax.numpy as jnp
from jax import lax
from jax.experimental import pallas as pl
from jax.experimental.pallas import tpu as pltpu
from jax.experimental.pallas import tpu_sc as plsc

_NUM_USERS = 20000
_NUM_ITEMS = 30000
_N = _NUM_USERS + _NUM_ITEMS
_D = 32
_G = 4
_B = 4096
_E = 1600000

_NC, _NS, _L = 2, 16, 16
_NW = _NC * _NS
_CHUNK = 128
_SUPER = 14
_RING = 6
_PF = 3
_NBLK = 392
_NSB = _NBLK // _SUPER
_EPAD = _NW * _NBLK * _CHUNK
_NPAD = 50176
_RPT = _NPAD // _NS
_MW = _NPAD // _NW
_KN = 392

_mesh = plsc.VectorSubcoreMesh(
    core_axis_name="c", subcore_axis_name="s", num_cores=_NC, num_subcores=_NS)
_sc_params = pltpu.CompilerParams(use_tc_tiling_on_sc=False)
_f32 = jnp.float32


def _wid():
    return lax.axis_index("c") * _NS + lax.axis_index("s")


def _splat(vec, lane):
    return vec.at[jnp.full((_L,), lane, jnp.int32)].get(
        mode="promise_in_bounds")


@functools.partial(
    pl.kernel,
    out_type=jax.ShapeDtypeStruct((_NC, _NPAD, _D), _f32),
    mesh=_mesh,
    compiler_params=_sc_params,
    scratch_types=[
        pltpu.VMEM((_SUPER, _CHUNK), jnp.int32),
        pltpu.VMEM((_SUPER, _CHUNK), jnp.int32),
        pltpu.VMEM((_SUPER, _CHUNK), _f32),
        pltpu.VMEM((_RING, _CHUNK, _D), _f32),
        pltpu.VMEM_SHARED((_NPAD, _D), _f32),
        pltpu.SemaphoreType.DMA((_RING,)),
        pltpu.SemaphoreType.DMA((_RING,)),
    ],
)
def _spmm(rows_hbm, cols_hbm, vals_hbm, table_hbm, zeros_hbm, out_hbm,
          idxr_v, idxc_v, vals_v, gbuf, acc, sem_g, sem_s):
    cid = lax.axis_index("c")
    sid = lax.axis_index("s")
    w = cid * _NS + sid

    r0 = sid * _RPT
    pltpu.sync_copy(zeros_hbm.at[pl.ds(r0, _RPT)], acc.at[pl.ds(r0, _RPT)])
    plsc.subcore_barrier()

    def _gather(k):
        return pltpu.make_async_copy(
            table_hbm.at[idxc_v.at[k]], gbuf.at[k % _RING],
            sem_g.at[k % _RING])

    def _scatter(k):
        return pltpu.make_async_copy(
            gbuf.at[k % _RING], acc.at[idxr_v.at[k]], sem_s.at[k % _RING])

    def superblock(sb, _):
        pltpu.sync_copy(rows_hbm.at[w, sb], idxr_v)
        pltpu.sync_copy(cols_hbm.at[w, sb], idxc_v)
        pltpu.sync_copy(vals_hbm.at[w, sb], vals_v)
        for k in range(_PF):
            _gather(k).start()
        for k in range(_SUPER):
            _gather(k).wait()

            def scale(grp, _, k=k):
                r = k % _RING
                vv = vals_v[k, pl.ds(grp * _L, _L)]
                for e2 in range(_L):
                    v = _splat(vv, e2)
                    e = grp * _L + e2
                    g0 = gbuf[r, e, pl.ds(0, _L)]
                    g1 = gbuf[r, e, pl.ds(_L, _L)]
                    gbuf[r, e, pl.ds(0, _L)] = g0 * v
                    gbuf[r, e, pl.ds(_L, _L)] = g1 * v
                return 0

            lax.fori_loop(0, _CHUNK // _L, scale, 0)
            _scatter(k).start(add=True)
            nk = k + _PF
            if nk < _SUPER:
                if nk >= _RING:
                    _scatter(nk - _RING).wait()
                _gather(nk).start()
        for k in range(max(0, _SUPER - _RING), _SUPER):
            _scatter(k).wait()
        return 0

    lax.fori_loop(0, _NSB, superblock, 0)

    plsc.subcore_barrier()
    pltpu.sync_copy(acc.at[pl.ds(r0, _RPT)], out_hbm.at[cid, pl.ds(r0, _RPT)])


@functools.partial(
    pl.kernel,
    out_type=[jax.ShapeDtypeStruct((_NPAD, _D), _f32)] * 5,
    mesh=_mesh,
    compiler_params=_sc_params,
    scratch_types=[
        pltpu.VMEM((_KN, _D), _f32),
        pltpu.VMEM((_KN, _D), _f32),
        pltpu.VMEM((_KN, _L), _f32),
        pltpu.VMEM((_KN, _D), _f32),
        pltpu.VMEM((_KN, _D), _f32),
        pltpu.VMEM((_KN, _D), _f32),
        pltpu.VMEM((_KN, _D), _f32),
    ],
)
def _mask1(p_hbm, oh_hbm, side_hbm, o0, o1, o2, o3, p0v, p1v, ohv,
           s0v, s1v, s2v, s3v):
    w = _wid()

    def chunk(c, _):
        n0 = w * _MW + c * _KN
        pltpu.sync_copy(p_hbm.at[0, pl.ds(n0, _KN)], p0v)
        pltpu.sync_copy(p_hbm.at[1, pl.ds(n0, _KN)], p1v)
        pltpu.sync_copy(oh_hbm.at[pl.ds(n0, _KN)], ohv)

        def node(n, _):
            a0 = p0v[n, pl.ds(0, _L)] + p1v[n, pl.ds(0, _L)]
            a1 = p0v[n, pl.ds(_L, _L)] + p1v[n, pl.ds(_L, _L)]
            p0v[n, pl.ds(0, _L)] = a0
            p0v[n, pl.ds(_L, _L)] = a1
            ov = ohv[n, pl.ds(0, _L)]
            for g, sv in enumerate((s0v, s1v, s2v, s3v)):
                m = _splat(ov, g)
                sv[n, pl.ds(0, _L)] = a0 * m
                sv[n, pl.ds(_L, _L)] = a1 * m
            return 0

        lax.fori_loop(0, _KN, node, 0)
        pltpu.sync_copy(p0v, side_hbm.at[pl.ds(n0, _KN)])
        for sv, oref in ((s0v, o0), (s1v, o1), (s2v, o2), (s3v, o3)):
            pltpu.sync_copy(sv, oref.at[pl.ds(n0, _KN)])
        return 0

    lax.fori_loop(0, _MW // _KN, chunk, 0)


@functools.partial(
    pl.kernel,
    out_type=[jax.ShapeDtypeStruct((_NPAD, _D), _f32)] * 5,
    mesh=_mesh,
    compiler_params=_sc_params,
    scratch_types=[
        [pltpu.VMEM((_KN, _D), _f32) for _ in range(8)],
        pltpu.VMEM((_KN, _L), _f32),
        pltpu.VMEM((_KN, _D), _f32),
        pltpu.SemaphoreType.DMA,
    ],
)
def _mask2(z0, z1, z2, z3, oh_hbm, l0, l1, l2, l3, ls_hbm, zv, ohv, lsv,
           sem):
    w = _wid()
    zrefs = (z0, z1, z2, z3)

    def chunk(c, _):
        n0 = w * _MW + c * _KN
        pltpu.sync_copy(oh_hbm.at[pl.ds(n0, _KN)], ohv)
        for g in range(_G):
            for p in range(2):
                pltpu.async_copy(zrefs[g].at[p, pl.ds(n0, _KN)],
                                 zv[2 * g + p], sem)
        for g in range(_G):
            for p in range(2):
                pltpu.make_async_copy(zrefs[g].at[p, pl.ds(n0, _KN)],
                                      zv[2 * g + p], sem).wait()

        def node(n, _):
            ov = ohv[n, pl.ds(0, _L)]
            r0 = jnp.zeros((_L,), _f32)
            r1 = jnp.zeros((_L,), _f32)
            for g in range(_G):
                m = _splat(ov, g)
                a0 = (zv[2 * g][n, pl.ds(0, _L)]
                      + zv[2 * g + 1][n, pl.ds(0, _L)]) * m
                a1 = (zv[2 * g][n, pl.ds(_L, _L)]
                      + zv[2 * g + 1][n, pl.ds(_L, _L)]) * m
                zv[2 * g][n, pl.ds(0, _L)] = a0
                zv[2 * g][n, pl.ds(_L, _L)] = a1
                r0 = r0 + a0
                r1 = r1 + a1
            lsv[n, pl.ds(0, _L)] = r0
            lsv[n, pl.ds(_L, _L)] = r1
            return 0

        lax.fori_loop(0, _KN, node, 0)
        for g, lout in enumerate((l0, l1, l2, l3)):
            pltpu.sync_copy(zv[2 * g], lout.at[pl.ds(n0, _KN)])
        pltpu.sync_copy(lsv, ls_hbm.at[pl.ds(n0, _KN)])
        return 0

    lax.fori_loop(0, _MW // _KN, chunk, 0)


@functools.partial(
    pl.kernel,
    out_type=[jax.ShapeDtypeStruct((_B, _D), _f32)] * 2,
    mesh=_mesh,
    compiler_params=_sc_params,
    scratch_types=[
        pltpu.VMEM((_CHUNK,), jnp.int32),
        pltpu.VMEM((_CHUNK, _D), _f32),
        pltpu.VMEM((_CHUNK, _D), _f32),
        [pltpu.VMEM((_CHUNK, _D), _f32) for _ in range(8)],
        pltpu.VMEM((_CHUNK, _L), _f32),
        pltpu.SemaphoreType.DMA,
    ],
)
def _final(side_hbm, ls_hbm, y0, y1, y2, y3, oh_hbm, uidx_hbm, iidx_hbm,
           u_hbm, v_hbm, idx_v, sv, lv, yv, ohv, sem):
    w = _wid()
    yrefs = (y0, y1, y2, y3)
    for idx_hbm, out_hbm in ((uidx_hbm, u_hbm), (iidx_hbm, v_hbm)):
        pltpu.sync_copy(idx_hbm.at[w, 0], idx_v)
        pltpu.async_copy(side_hbm.at[idx_v], sv, sem)
        pltpu.async_copy(ls_hbm.at[idx_v], lv, sem)
        pltpu.async_copy(oh_hbm.at[idx_v], ohv, sem)
        for g in range(_G):
            for p in range(2):
                pltpu.async_copy(yrefs[g].at[p].at[idx_v], yv[2 * g + p],
                                 sem)
        pltpu.make_async_copy(side_hbm.at[idx_v], sv, sem).wait()
        pltpu.make_async_copy(ls_hbm.at[idx_v], lv, sem).wait()
        pltpu.make_async_copy(oh_hbm.at[idx_v], ohv, sem).wait()
        for g in range(_G):
            for p in range(2):
                pltpu.make_async_copy(yrefs[g].at[p].at[idx_v],
                                      yv[2 * g + p], sem).wait()

        def node(n, _):
            ov = ohv[n, pl.ds(0, _L)]
            for h in range(2):
                acc = 4.0 * sv[n, pl.ds(h * _L, _L)] + lv[n, pl.ds(h * _L,
                                                                   _L)]
                for g in range(_G):
                    m = _splat(ov, g)
                    acc = acc + m * (yv[2 * g][n, pl.ds(h * _L, _L)]
                                     + yv[2 * g + 1][n, pl.ds(h * _L, _L)])
                sv[n, pl.ds(h * _L, _L)] = 0.2 * acc
            return 0

        lax.fori_loop(0, _CHUNK, node, 0)
        pltpu.sync_copy(sv, out_hbm.at[pl.ds(w * _CHUNK, _CHUNK)])


_BLK = 6272


def _dense1_body(emb_ref, p_ref, fcw_ref, fcb_ref, fgw_ref, fgb_ref, oh_ref):
    side = p_ref[0] + p_ref[1]
    x = emb_ref[...] + side
    t = jnp.dot(x, fcw_ref[...], preferred_element_type=_f32)
    t = t + fcb_ref[...]
    t = jnp.where(t >= 0, t, 0.01 * t)
    sc = jnp.dot(t, fgw_ref[...], preferred_element_type=_f32)
    sc = sc + fgb_ref[...]
    amax = jnp.max(sc, axis=1, keepdims=True)
    rows = pl.program_id(0) * _BLK + lax.broadcasted_iota(
        jnp.int32, (_BLK, 1), 0)
    oh = jnp.where(rows < _NUM_USERS, (sc == amax).astype(_f32), _f32(1.0))
    oh_ref[...] = jnp.concatenate([oh, oh, oh, oh], axis=1)


def _dense1(all_emb, partials, fc_W, fc_b, fcg_W, fcg_b):
    return pl.pallas_call(
        _dense1_body,
        grid=(_NPAD // _BLK,),
        in_specs=[
            pl.BlockSpec((_BLK, _D), lambda i: (i, 0)),
            pl.BlockSpec((_NC, _BLK, _D), lambda i: (0, i, 0)),
            pl.BlockSpec((_D, _D), lambda i: (0, 0)),
            pl.BlockSpec((1, _D), lambda i: (0, 0)),
            pl.BlockSpec((_D, _G), lambda i: (0, 0)),
            pl.BlockSpec((1, _G), lambda i: (0, 0)),
        ],
        out_specs=pl.BlockSpec((_BLK, _L), lambda i: (i, 0)),
        out_shape=jax.ShapeDtypeStruct((_NPAD, _L), _f32),
    )(all_emb, partials, fc_W, fc_b[None, :], fcg_W, fcg_b[None, :])


def _dot_body(u_ref, v_ref, o_ref):
    o_ref[...] = jnp.sum(u_ref[...] * v_ref[...], axis=1, keepdims=True)


def _rowdot(u, v):
    return pl.pallas_call(
        _dot_body,
        grid=(1,),
        in_specs=[pl.BlockSpec((_B, _D), lambda i: (0, 0))] * 2,
        out_specs=pl.BlockSpec((_B, 1), lambda i: (0, 0)),
        out_shape=jax.ShapeDtypeStruct((_B, 1), _f32),
    )(u, v)


def kernel(users, items, edge_index, graph_vals, emb_user, emb_item,
           fc_W, fc_b, fcg_W, fcg_b):
    all_emb = jnp.concatenate([emb_user, emb_item], axis=0)
    all_emb = jnp.pad(all_emb, ((0, _NPAD - _N), (0, 0)))

    pad = _EPAD - _E
    rows = jnp.pad(edge_index[0], (0, pad)).reshape(_NW, _NSB, _SUPER, _CHUNK)
    cols = jnp.pad(edge_index[1], (0, pad)).reshape(_NW, _NSB, _SUPER, _CHUNK)
    vals = jnp.pad(graph_vals, (0, pad)).reshape(_NW, _NSB, _SUPER, _CHUNK)
    zeros = jnp.zeros((_NPAD, _D), _f32)

    p_side = _spmm(rows, cols, vals, all_emb, zeros)
    oh16 = _dense1(all_emb, p_side, fc_W, fc_b, fcg_W, fcg_b)
    side, s0, s1, s2, s3 = _mask1(p_side, oh16)
    z = [_spmm(rows, cols, vals, s, zeros) for s in (s0, s1, s2, s3)]
    l0, l1, l2, l3, l1sum = _mask2(*z, oh16)
    y = [_spmm(rows, cols, vals, t, zeros) for t in (l0, l1, l2, l3)]

    uidx = users.astype(jnp.int32).reshape(_NW, 1, _CHUNK)
    iidx = (items.astype(jnp.int32) + _NUM_USERS).reshape(_NW, 1, _CHUNK)
    u, v = _final(side, l1sum, *y, oh16, uidx, iidx)
    return _rowdot(u, v).reshape(_B)


# speedup vs baseline: 1.3321x; 1.0354x over previous
import functools

import jax
import jax.numpy as jnp
from jax import lax
from jax.experimental import pallas as pl
from jax.experimental.pallas import tpu as pltpu
from jax.experimental.pallas import tpu_sc as plsc

_NUM_USERS = 20000
_NUM_ITEMS = 30000
_N = _NUM_USERS + _NUM_ITEMS
_D = 32
_G = 4
_B = 4096
_E = 1600000

_NC, _NS, _L = 2, 16, 16
_NW = _NC * _NS
_CHUNK = 128
_SUPER = 14
_RING = 6
_PF = 4
_NBLK = 392
_NSB = _NBLK // _SUPER
_EPAD = _NW * _NBLK * _CHUNK
_NPAD = 50176
_RPT = _NPAD // _NS
_MW = _NPAD // _NW
_KN = 392

_mesh = plsc.VectorSubcoreMesh(
    core_axis_name="c", subcore_axis_name="s", num_cores=_NC, num_subcores=_NS)
_sc_params = pltpu.CompilerParams(use_tc_tiling_on_sc=False)
_f32 = jnp.float32


def _wid():
    return lax.axis_index("c") * _NS + lax.axis_index("s")


def _splat(vec, lane):
    return vec.at[jnp.full((_L,), lane, jnp.int32)].get(
        mode="promise_in_bounds")


@functools.partial(
    pl.kernel,
    out_type=jax.ShapeDtypeStruct((_NC, _NPAD, _D), _f32),
    mesh=_mesh,
    compiler_params=_sc_params,
    scratch_types=[
        pltpu.VMEM((_SUPER, _CHUNK), jnp.int32),
        pltpu.VMEM((_SUPER, _CHUNK), jnp.int32),
        pltpu.VMEM((_SUPER, _CHUNK), _f32),
        pltpu.VMEM((_RING, _CHUNK, _D), _f32),
        pltpu.VMEM_SHARED((_NPAD, _D), _f32),
        pltpu.SemaphoreType.DMA((_RING,)),
        pltpu.SemaphoreType.DMA((_RING,)),
    ],
)
def _spmm(rows_hbm, cols_hbm, vals_hbm, table_hbm, zeros_hbm, out_hbm,
          idxr_v, idxc_v, vals_v, gbuf, acc, sem_g, sem_s):
    cid = lax.axis_index("c")
    sid = lax.axis_index("s")
    w = cid * _NS + sid

    r0 = sid * _RPT
    pltpu.sync_copy(zeros_hbm.at[pl.ds(r0, _RPT)], acc.at[pl.ds(r0, _RPT)])
    plsc.subcore_barrier()

    def _gather(k):
        return pltpu.make_async_copy(
            table_hbm.at[idxc_v.at[k]], gbuf.at[k % _RING],
            sem_g.at[k % _RING])

    def _scatter(k):
        return pltpu.make_async_copy(
            gbuf.at[k % _RING], acc.at[idxr_v.at[k]], sem_s.at[k % _RING])

    def superblock(sb, _):
        pltpu.sync_copy(rows_hbm.at[w, sb], idxr_v)
        pltpu.sync_copy(cols_hbm.at[w, sb], idxc_v)
        pltpu.sync_copy(vals_hbm.at[w, sb], vals_v)
        for k in range(_PF):
            _gather(k).start()
        for k in range(_SUPER):
            _gather(k).wait()

            def scale(grp, _, k=k):
                r = k % _RING
                vv = vals_v[k, pl.ds(grp * _L, _L)]
                for e2 in range(_L):
                    v = _splat(vv, e2)
                    e = grp * _L + e2
                    g0 = gbuf[r, e, pl.ds(0, _L)]
                    g1 = gbuf[r, e, pl.ds(_L, _L)]
                    gbuf[r, e, pl.ds(0, _L)] = g0 * v
                    gbuf[r, e, pl.ds(_L, _L)] = g1 * v
                return 0

            lax.fori_loop(0, _CHUNK // _L, scale, 0)
            _scatter(k).start(add=True)
            nk = k + _PF
            if nk < _SUPER:
                if nk >= _RING:
                    _scatter(nk - _RING).wait()
                _gather(nk).start()
        for k in range(max(0, _SUPER - _RING), _SUPER):
            _scatter(k).wait()
        return 0

    lax.fori_loop(0, _NSB, superblock, 0)

    plsc.subcore_barrier()
    pltpu.sync_copy(acc.at[pl.ds(r0, _RPT)], out_hbm.at[cid, pl.ds(r0, _RPT)])


@functools.partial(
    pl.kernel,
    out_type=[jax.ShapeDtypeStruct((_NPAD, _D), _f32)] * 5,
    mesh=_mesh,
    compiler_params=_sc_params,
    scratch_types=[
        pltpu.VMEM((_KN, _D), _f32),
        pltpu.VMEM((_KN, _D), _f32),
        pltpu.VMEM((_KN, _L), _f32),
        pltpu.VMEM((_KN, _D), _f32),
        pltpu.VMEM((_KN, _D), _f32),
        pltpu.VMEM((_KN, _D), _f32),
        pltpu.VMEM((_KN, _D), _f32),
    ],
)
def _mask1(p_hbm, oh_hbm, side_hbm, o0, o1, o2, o3, p0v, p1v, ohv,
           s0v, s1v, s2v, s3v):
    w = _wid()

    def chunk(c, _):
        n0 = w * _MW + c * _KN
        pltpu.sync_copy(p_hbm.at[0, pl.ds(n0, _KN)], p0v)
        pltpu.sync_copy(p_hbm.at[1, pl.ds(n0, _KN)], p1v)
        pltpu.sync_copy(oh_hbm.at[pl.ds(n0, _KN)], ohv)

        def node(n, _):
            a0 = p0v[n, pl.ds(0, _L)] + p1v[n, pl.ds(0, _L)]
            a1 = p0v[n, pl.ds(_L, _L)] + p1v[n, pl.ds(_L, _L)]
            p0v[n, pl.ds(0, _L)] = a0
            p0v[n, pl.ds(_L, _L)] = a1
            ov = ohv[n, pl.ds(0, _L)]
            for g, sv in enumerate((s0v, s1v, s2v, s3v)):
                m = _splat(ov, g)
                sv[n, pl.ds(0, _L)] = a0 * m
                sv[n, pl.ds(_L, _L)] = a1 * m
            return 0

        lax.fori_loop(0, _KN, node, 0)
        pltpu.sync_copy(p0v, side_hbm.at[pl.ds(n0, _KN)])
        for sv, oref in ((s0v, o0), (s1v, o1), (s2v, o2), (s3v, o3)):
            pltpu.sync_copy(sv, oref.at[pl.ds(n0, _KN)])
        return 0

    lax.fori_loop(0, _MW // _KN, chunk, 0)


@functools.partial(
    pl.kernel,
    out_type=[jax.ShapeDtypeStruct((_NPAD, _D), _f32)] * 5,
    mesh=_mesh,
    compiler_params=_sc_params,
    scratch_types=[
        [pltpu.VMEM((_KN, _D), _f32) for _ in range(8)],
        pltpu.VMEM((_KN, _L), _f32),
        pltpu.VMEM((_KN, _D), _f32),
        pltpu.SemaphoreType.DMA,
    ],
)
def _mask2(z0, z1, z2, z3, oh_hbm, l0, l1, l2, l3, ls_hbm, zv, ohv, lsv,
           sem):
    w = _wid()
    zrefs = (z0, z1, z2, z3)

    def chunk(c, _):
        n0 = w * _MW + c * _KN
        pltpu.sync_copy(oh_hbm.at[pl.ds(n0, _KN)], ohv)
        for g in range(_G):
            for p in range(2):
                pltpu.async_copy(zrefs[g].at[p, pl.ds(n0, _KN)],
                                 zv[2 * g + p], sem)
        for g in range(_G):
            for p in range(2):
                pltpu.make_async_copy(zrefs[g].at[p, pl.ds(n0, _KN)],
                                      zv[2 * g + p], sem).wait()

        def node(n, _):
            ov = ohv[n, pl.ds(0, _L)]
            r0 = jnp.zeros((_L,), _f32)
            r1 = jnp.zeros((_L,), _f32)
            for g in range(_G):
                m = _splat(ov, g)
                a0 = (zv[2 * g][n, pl.ds(0, _L)]
                      + zv[2 * g + 1][n, pl.ds(0, _L)]) * m
                a1 = (zv[2 * g][n, pl.ds(_L, _L)]
                      + zv[2 * g + 1][n, pl.ds(_L, _L)]) * m
                zv[2 * g][n, pl.ds(0, _L)] = a0
                zv[2 * g][n, pl.ds(_L, _L)] = a1
                r0 = r0 + a0
                r1 = r1 + a1
            lsv[n, pl.ds(0, _L)] = r0
            lsv[n, pl.ds(_L, _L)] = r1
            return 0

        lax.fori_loop(0, _KN, node, 0)
        for g, lout in enumerate((l0, l1, l2, l3)):
            pltpu.sync_copy(zv[2 * g], lout.at[pl.ds(n0, _KN)])
        pltpu.sync_copy(lsv, ls_hbm.at[pl.ds(n0, _KN)])
        return 0

    lax.fori_loop(0, _MW // _KN, chunk, 0)


@functools.partial(
    pl.kernel,
    out_type=[jax.ShapeDtypeStruct((_B, _D), _f32)] * 2,
    mesh=_mesh,
    compiler_params=_sc_params,
    scratch_types=[
        pltpu.VMEM((_CHUNK,), jnp.int32),
        pltpu.VMEM((_CHUNK, _D), _f32),
        pltpu.VMEM((_CHUNK, _D), _f32),
        [pltpu.VMEM((_CHUNK, _D), _f32) for _ in range(8)],
        pltpu.VMEM((_CHUNK, _L), _f32),
        pltpu.SemaphoreType.DMA,
    ],
)
def _final(side_hbm, ls_hbm, y0, y1, y2, y3, oh_hbm, uidx_hbm, iidx_hbm,
           u_hbm, v_hbm, idx_v, sv, lv, yv, ohv, sem):
    w = _wid()
    yrefs = (y0, y1, y2, y3)
    for idx_hbm, out_hbm in ((uidx_hbm, u_hbm), (iidx_hbm, v_hbm)):
        pltpu.sync_copy(idx_hbm.at[w, 0], idx_v)
        pltpu.async_copy(side_hbm.at[idx_v], sv, sem)
        pltpu.async_copy(ls_hbm.at[idx_v], lv, sem)
        pltpu.async_copy(oh_hbm.at[idx_v], ohv, sem)
        for g in range(_G):
            for p in range(2):
                pltpu.async_copy(yrefs[g].at[p].at[idx_v], yv[2 * g + p],
                                 sem)
        pltpu.make_async_copy(side_hbm.at[idx_v], sv, sem).wait()
        pltpu.make_async_copy(ls_hbm.at[idx_v], lv, sem).wait()
        pltpu.make_async_copy(oh_hbm.at[idx_v], ohv, sem).wait()
        for g in range(_G):
            for p in range(2):
                pltpu.make_async_copy(yrefs[g].at[p].at[idx_v],
                                      yv[2 * g + p], sem).wait()

        def node(n, _):
            ov = ohv[n, pl.ds(0, _L)]
            for h in range(2):
                acc = 4.0 * sv[n, pl.ds(h * _L, _L)] + lv[n, pl.ds(h * _L,
                                                                   _L)]
                for g in range(_G):
                    m = _splat(ov, g)
                    acc = acc + m * (yv[2 * g][n, pl.ds(h * _L, _L)]
                                     + yv[2 * g + 1][n, pl.ds(h * _L, _L)])
                sv[n, pl.ds(h * _L, _L)] = 0.2 * acc
            return 0

        lax.fori_loop(0, _CHUNK, node, 0)
        pltpu.sync_copy(sv, out_hbm.at[pl.ds(w * _CHUNK, _CHUNK)])


_BLK = 6272


def _dense1_body(emb_ref, p_ref, fcw_ref, fcb_ref, fgw_ref, fgb_ref, oh_ref):
    side = p_ref[0] + p_ref[1]
    x = emb_ref[...] + side
    t = jnp.dot(x, fcw_ref[...], preferred_element_type=_f32)
    t = t + fcb_ref[...]
    t = jnp.where(t >= 0, t, 0.01 * t)
    sc = jnp.dot(t, fgw_ref[...], preferred_element_type=_f32)
    sc = sc + fgb_ref[...]
    amax = jnp.max(sc, axis=1, keepdims=True)
    rows = pl.program_id(0) * _BLK + lax.broadcasted_iota(
        jnp.int32, (_BLK, 1), 0)
    oh = jnp.where(rows < _NUM_USERS, (sc == amax).astype(_f32), _f32(1.0))
    oh_ref[...] = jnp.concatenate([oh, oh, oh, oh], axis=1)


def _dense1(all_emb, partials, fc_W, fc_b, fcg_W, fcg_b):
    return pl.pallas_call(
        _dense1_body,
        grid=(_NPAD // _BLK,),
        in_specs=[
            pl.BlockSpec((_BLK, _D), lambda i: (i, 0)),
            pl.BlockSpec((_NC, _BLK, _D), lambda i: (0, i, 0)),
            pl.BlockSpec((_D, _D), lambda i: (0, 0)),
            pl.BlockSpec((1, _D), lambda i: (0, 0)),
            pl.BlockSpec((_D, _G), lambda i: (0, 0)),
            pl.BlockSpec((1, _G), lambda i: (0, 0)),
        ],
        out_specs=pl.BlockSpec((_BLK, _L), lambda i: (i, 0)),
        out_shape=jax.ShapeDtypeStruct((_NPAD, _L), _f32),
    )(all_emb, partials, fc_W, fc_b[None, :], fcg_W, fcg_b[None, :])


def _dot_body(u_ref, v_ref, o_ref):
    o_ref[...] = jnp.sum(u_ref[...] * v_ref[...], axis=1, keepdims=True)


def _rowdot(u, v):
    return pl.pallas_call(
        _dot_body,
        grid=(1,),
        in_specs=[pl.BlockSpec((_B, _D), lambda i: (0, 0))] * 2,
        out_specs=pl.BlockSpec((_B, 1), lambda i: (0, 0)),
        out_shape=jax.ShapeDtypeStruct((_B, 1), _f32),
    )(u, v)


def kernel(users, items, edge_index, graph_vals, emb_user, emb_item,
           fc_W, fc_b, fcg_W, fcg_b):
    all_emb = jnp.concatenate([emb_user, emb_item], axis=0)
    all_emb = jnp.pad(all_emb, ((0, _NPAD - _N), (0, 0)))

    pad = _EPAD - _E
    rows = jnp.pad(edge_index[0], (0, pad)).reshape(_NW, _NSB, _SUPER, _CHUNK)
    cols = jnp.pad(edge_index[1], (0, pad)).reshape(_NW, _NSB, _SUPER, _CHUNK)
    vals = jnp.pad(graph_vals, (0, pad)).reshape(_NW, _NSB, _SUPER, _CHUNK)
    zeros = jnp.zeros((_NPAD, _D), _f32)

    p_side = _spmm(rows, cols, vals, all_emb, zeros)
    oh16 = _dense1(all_emb, p_side, fc_W, fc_b, fcg_W, fcg_b)
    side, s0, s1, s2, s3 = _mask1(p_side, oh16)
    z = [_spmm(rows, cols, vals, s, zeros) for s in (s0, s1, s2, s3)]
    l0, l1, l2, l3, l1sum = _mask2(*z, oh16)
    y = [_spmm(rows, cols, vals, t, zeros) for t in (l0, l1, l2, l3)]

    uidx = users.astype(jnp.int32).reshape(_NW, 1, _CHUNK)
    iidx = (items.astype(jnp.int32) + _NUM_USERS).reshape(_NW, 1, _CHUNK)
    u, v = _final(side, l1sum, *y, oh16, uidx, iidx)
    return _rowdot(u, v).reshape(_B)
